# baseline (speedup 1.0000x reference)
import functools

import jax
import jax.numpy as jnp
from jax import lax
from jax.experimental import pallas as pl
from jax.experimental.pallas import tpu as pltpu
from jax.experimental.pallas import tpu_sc as plsc

N = 10000
E = 320000
D = 128

NC = 2
NS = 16
NW = NC * NS

E_PER_W = E // NW
CHUNK = 80
NCHUNK = E_PER_W // CHUNK
NPAD = 10240
RPT = NPAD // NS
ZROWS = 16
NBUF = 4
IBUF = 2 * NBUF


def _sc_aggregate(x, edge3):
  mesh = plsc.VectorSubcoreMesh(core_axis_name="c", subcore_axis_name="s")

  @functools.partial(
      pl.kernel,
      out_type=(
          jax.ShapeDtypeStruct((NC, NPAD, D), jnp.float32),
          jax.ShapeDtypeStruct((NC, NPAD), jnp.float32),
      ),
      mesh=mesh,
      compiler_params=pltpu.CompilerParams(use_tc_tiling_on_sc=False),
      scratch_types=[
          [pltpu.VMEM((2, CHUNK), jnp.int32) for _ in range(IBUF)],
          [pltpu.VMEM((CHUNK, D), jnp.float32) for _ in range(NBUF)],
          pltpu.VMEM((CHUNK,), jnp.float32),
          pltpu.VMEM((ZROWS, D), jnp.float32),
          pltpu.VMEM((RPT,), jnp.float32),
          pltpu.VMEM_SHARED((NPAD, D), jnp.float32),
          pltpu.VMEM_SHARED((NPAD,), jnp.float32),
          [pltpu.SemaphoreType.DMA for _ in range(IBUF)],
          [pltpu.SemaphoreType.DMA for _ in range(NBUF)],
          pltpu.SemaphoreType.DMA,
          pltpu.SemaphoreType.DMA,
      ],
  )
  def sc_kernel(x_hbm, e_hbm, agg_out, cnt_out,
                ibufs, rows, ones, zbuf, czbuf, agg_sh, cnt_sh,
                semi, semg, sems, semc):
    c = lax.axis_index("c")
    s = lax.axis_index("s")
    wid = s * NC + c

    def zrow(i, _):
      zbuf[i // 8, pl.ds((i % 8) * 16, 16)] = jnp.zeros((16,), jnp.float32)
      return 0
    lax.fori_loop(0, ZROWS * (D // 16), zrow, 0)

    def czrow(i, _):
      czbuf[pl.ds(i * 16, 16)] = jnp.zeros((16,), jnp.float32)
      return 0
    lax.fori_loop(0, RPT // 16, czrow, 0)

    def onerow(i, _):
      ones[pl.ds(i * 16, 16)] = jnp.ones((16,), jnp.float32)
      return 0
    lax.fori_loop(0, CHUNK // 16, onerow, 0)

    base = s * RPT
    for k in range(RPT // ZROWS):
      pltpu.sync_copy(zbuf, agg_sh.at[pl.ds(base + k * ZROWS, ZROWS)])
    pltpu.sync_copy(czbuf, cnt_sh.at[pl.ds(base, RPT)])
    plsc.subcore_barrier()

    for k in range(IBUF):
      pltpu.async_copy(e_hbm.at[wid, k], ibufs[k], semi[k])
    for k in range(NBUF):
      pltpu.make_async_copy(e_hbm.at[wid, k], ibufs[k], semi[k]).wait()
      pltpu.async_copy(x_hbm.at[ibufs[k].at[0]], rows[k], semg[k])

    def step(i, v):
      b = v % NBUF
      gslot = (v + NBUF) % IBUF
      ib = ibufs[v]
      pltpu.make_async_copy(x_hbm.at[ib.at[0]], rows[b], semg[b]).wait()
      sa = pltpu.async_copy(rows[b], agg_sh.at[ib.at[1]], sems, add=True)
      sc = pltpu.async_copy(ones, cnt_sh.at[ib.at[1]], semc, add=True)
      sa.wait()
      sc.wait()
      nxt_load = jnp.minimum(i + IBUF, NCHUNK - 1)
      pltpu.async_copy(e_hbm.at[wid, nxt_load], ib, semi[v])
      pltpu.make_async_copy(e_hbm.at[wid, 0], ibufs[gslot], semi[gslot]).wait()
      pltpu.async_copy(x_hbm.at[ibufs[gslot].at[0]], rows[b], semg[b])

    def chunk_body(i, _):
      for v in range(IBUF):
        @pl.when(i % IBUF == v)
        def _():
          step(i, v)
      return 0
    lax.fori_loop(0, NCHUNK, chunk_body, 0)
    for b in range(NBUF):
      pltpu.make_async_copy(x_hbm.at[ibufs[0].at[0]], rows[b], semg[b]).wait()
    issues = [1 + len([i for i in range(NCHUNK) if i % IBUF == k])
              for k in range(IBUF)]
    waits = [(1 if k < NBUF else 0)
             + len([i for i in range(NCHUNK) if (i + NBUF) % IBUF == k])
             for k in range(IBUF)]
    for k in range(IBUF):
      for _ in range(issues[k] - waits[k]):
        pltpu.make_async_copy(e_hbm.at[wid, 0], ibufs[k], semi[k]).wait()
    plsc.subcore_barrier()

    for k in range(RPT // ZROWS):
      pltpu.sync_copy(agg_sh.at[pl.ds(base + k * ZROWS, ZROWS)], zbuf)
      pltpu.sync_copy(zbuf, agg_out.at[c, pl.ds(base + k * ZROWS, ZROWS)])
    pltpu.sync_copy(cnt_sh.at[pl.ds(base, RPT)], czbuf)
    pltpu.sync_copy(czbuf, cnt_out.at[c, pl.ds(base, RPT)])

  return sc_kernel(x, edge3)


def _tc_finish_body(agg_ref, cnt_ref, x_ref, wl_ref, bl_ref, wr_ref,
                    g_ref, b_ref, out_ref):
  agg = agg_ref[0, :N] + agg_ref[1, :N]
  cnt = cnt_ref[0, :N] + cnt_ref[1, :N]
  mean = agg / jnp.clip(cnt, 1.0, None)[:, None]
  h = (jnp.dot(mean, wl_ref[...].T, preferred_element_type=jnp.float32)
       + bl_ref[...][None, :]
       + jnp.dot(x_ref[...], wr_ref[...].T, preferred_element_type=jnp.float32))
  h = jnp.maximum(h, 0.0)
  mu = jnp.mean(h, axis=0)
  var = jnp.mean((h - mu[None, :]) ** 2, axis=0)
  out_ref[...] = ((h - mu[None, :]) * lax.rsqrt(var + 1e-5)
                  * g_ref[...][None, :] + b_ref[...][None, :])


def kernel(x, edge_index, W_l, b_l, W_r, gamma, beta):
  edge3 = jnp.stack(
      [edge_index[0].reshape(NW, NCHUNK, CHUNK),
       edge_index[1].reshape(NW, NCHUNK, CHUNK)], axis=2)
  agg_p, cnt_p = _sc_aggregate(x, edge3)
  return pl.pallas_call(
      _tc_finish_body,
      out_shape=jax.ShapeDtypeStruct((N, D), jnp.float32),
  )(agg_p, cnt_p, x, W_l, b_l, W_r, gamma, beta)


# speedup vs baseline: 1.1041x; 1.1041x over previous
import functools

import jax
import jax.numpy as jnp
from jax import lax
from jax.experimental import pallas as pl
from jax.experimental.pallas import tpu as pltpu
from jax.experimental.pallas import tpu_sc as plsc

N = 10000
E = 320000
D = 128

NC = 2
NS = 16
NW = NC * NS

E_PER_W = E // NW
CHUNK = 80
NCHUNK = E_PER_W // CHUNK
NPAD = 10240
RPT = NPAD // NS
ZROWS = 32


def _sc_aggregate(xb, src3, dst3):
  mesh = plsc.VectorSubcoreMesh(core_axis_name="c", subcore_axis_name="s")

  @functools.partial(
      pl.kernel,
      out_type=(
          jax.ShapeDtypeStruct((NC, NPAD, D), jnp.bfloat16),
          jax.ShapeDtypeStruct((NC, NPAD), jnp.float32),
      ),
      mesh=mesh,
      compiler_params=pltpu.CompilerParams(use_tc_tiling_on_sc=False),
      scratch_types=[
          pltpu.VMEM((NCHUNK, CHUNK), jnp.int32),
          pltpu.VMEM((NCHUNK, CHUNK), jnp.int32),
          pltpu.VMEM((CHUNK, D), jnp.bfloat16),
          pltpu.VMEM((CHUNK, D), jnp.bfloat16),
          pltpu.VMEM((CHUNK,), jnp.float32),
          pltpu.VMEM((ZROWS, D), jnp.bfloat16),
          pltpu.VMEM((RPT,), jnp.float32),
          pltpu.VMEM_SHARED((NPAD, D), jnp.bfloat16),
          pltpu.VMEM_SHARED((NPAD,), jnp.float32),
          pltpu.SemaphoreType.DMA,
          pltpu.SemaphoreType.DMA,
          pltpu.SemaphoreType.DMA,
          pltpu.SemaphoreType.DMA,
      ],
  )
  def sc_kernel(x_hbm, src_hbm, dst_hbm, agg_out, cnt_out,
                srcl, dstl, rows0, rows1, ones, zbuf, czbuf, agg_sh, cnt_sh,
                semg0, semg1, sems, semc):
    c = lax.axis_index("c")
    s = lax.axis_index("s")
    wid = s * NC + c

    def zrow(i, _):
      zbuf[i // 4, pl.ds((i % 4) * 32, 32)] = jnp.zeros((32,), jnp.bfloat16)
      return 0
    lax.fori_loop(0, ZROWS * (D // 32), zrow, 0)

    def czrow(i, _):
      czbuf[pl.ds(i * 16, 16)] = jnp.zeros((16,), jnp.float32)
      return 0
    lax.fori_loop(0, RPT // 16, czrow, 0)

    def onerow(i, _):
      ones[pl.ds(i * 16, 16)] = jnp.ones((16,), jnp.float32)
      return 0
    lax.fori_loop(0, CHUNK // 16, onerow, 0)

    base = s * RPT
    for k in range(RPT // ZROWS):
      pltpu.sync_copy(zbuf, agg_sh.at[pl.ds(base + k * ZROWS, ZROWS)])
    pltpu.sync_copy(czbuf, cnt_sh.at[pl.ds(base, RPT)])
    plsc.subcore_barrier()

    pltpu.sync_copy(src_hbm.at[wid], srcl)
    pltpu.sync_copy(dst_hbm.at[wid], dstl)

    pltpu.async_copy(x_hbm.at[srcl.at[0]], rows0, semg0)
    pltpu.async_copy(x_hbm.at[srcl.at[1]], rows1, semg1)

    def step(i, rows_b, semg_b):
      pltpu.make_async_copy(x_hbm.at[srcl.at[i]], rows_b, semg_b).wait()
      sa = pltpu.async_copy(rows_b, agg_sh.at[dstl.at[i]], sems, add=True)
      sc = pltpu.async_copy(ones, cnt_sh.at[dstl.at[i]], semc, add=True)
      sa.wait()
      sc.wait()
      nxt = jnp.minimum(i + 2, NCHUNK - 1)
      pltpu.async_copy(x_hbm.at[srcl.at[nxt]], rows_b, semg_b)

    def chunk_body(i, _):
      @pl.when(i % 2 == 0)
      def _():
        step(i, rows0, semg0)
      @pl.when(i % 2 == 1)
      def _():
        step(i, rows1, semg1)
      return 0
    lax.fori_loop(0, NCHUNK, chunk_body, 0)
    pltpu.make_async_copy(x_hbm.at[srcl.at[NCHUNK - 1]], rows0, semg0).wait()
    pltpu.make_async_copy(x_hbm.at[srcl.at[NCHUNK - 1]], rows1, semg1).wait()
    plsc.subcore_barrier()

    for k in range(RPT // ZROWS):
      pltpu.sync_copy(agg_sh.at[pl.ds(base + k * ZROWS, ZROWS)], zbuf)
      pltpu.sync_copy(zbuf, agg_out.at[c, pl.ds(base + k * ZROWS, ZROWS)])
    pltpu.sync_copy(cnt_sh.at[pl.ds(base, RPT)], czbuf)
    pltpu.sync_copy(czbuf, cnt_out.at[c, pl.ds(base, RPT)])

  return sc_kernel(xb, src3, dst3)


def _tc_finish_body(agg_ref, cnt_ref, x_ref, wl_ref, bl_ref, wr_ref,
                    g_ref, b_ref, out_ref):
  agg = (agg_ref[0, :N].astype(jnp.float32)
         + agg_ref[1, :N].astype(jnp.float32))
  cnt = cnt_ref[0, :N] + cnt_ref[1, :N]
  mean = agg / jnp.clip(cnt, 1.0, None)[:, None]
  h = (jnp.dot(mean, wl_ref[...].T, preferred_element_type=jnp.float32)
       + bl_ref[...][None, :]
       + jnp.dot(x_ref[...], wr_ref[...].T, preferred_element_type=jnp.float32))
  h = jnp.maximum(h, 0.0)
  mu = jnp.mean(h, axis=0)
  var = jnp.mean((h - mu[None, :]) ** 2, axis=0)
  out_ref[...] = ((h - mu[None, :]) * lax.rsqrt(var + 1e-5)
                  * g_ref[...][None, :] + b_ref[...][None, :])


def kernel(x, edge_index, W_l, b_l, W_r, gamma, beta):
  xb = x.astype(jnp.bfloat16)
  src3 = edge_index[0].reshape(NW, NCHUNK, CHUNK)
  dst3 = edge_index[1].reshape(NW, NCHUNK, CHUNK)
  agg_p, cnt_p = _sc_aggregate(xb, src3, dst3)
  return pl.pallas_call(
      _tc_finish_body,
      out_shape=jax.ShapeDtypeStruct((N, D), jnp.float32),
  )(agg_p, cnt_p, x, W_l, b_l, W_r, gamma, beta)


# speedup vs baseline: 1.1051x; 1.0009x over previous
import functools

import jax
import jax.numpy as jnp
from jax import lax
from jax.experimental import pallas as pl
from jax.experimental.pallas import tpu as pltpu
from jax.experimental.pallas import tpu_sc as plsc

N = 10000
E = 320000
D = 128

NC = 2
NS = 16
NW = NC * NS

E_PER_W = E // NW
CHUNK = 80
NCHUNK = E_PER_W // CHUNK
NPAD = 10240
RPT = NPAD // NS
ZROWS = 128
NZB = RPT // ZROWS


def _sc_aggregate(xb, src3, dst3, zrows_c, zcnt_c, ones_c):
  mesh = plsc.VectorSubcoreMesh(core_axis_name="c", subcore_axis_name="s")

  @functools.partial(
      pl.kernel,
      out_type=(
          jax.ShapeDtypeStruct((NC, NPAD, D), jnp.bfloat16),
          jax.ShapeDtypeStruct((NC, NPAD), jnp.float32),
      ),
      mesh=mesh,
      compiler_params=pltpu.CompilerParams(use_tc_tiling_on_sc=False),
      scratch_types=[
          pltpu.VMEM((NCHUNK, CHUNK), jnp.int32),
          pltpu.VMEM((NCHUNK, CHUNK), jnp.int32),
          pltpu.VMEM((CHUNK, D), jnp.bfloat16),
          pltpu.VMEM((CHUNK, D), jnp.bfloat16),
          pltpu.VMEM((CHUNK,), jnp.float32),
          pltpu.VMEM((ZROWS, D), jnp.bfloat16),
          pltpu.VMEM((ZROWS, D), jnp.bfloat16),
          pltpu.VMEM((RPT,), jnp.float32),
          pltpu.VMEM_SHARED((NPAD, D), jnp.bfloat16),
          pltpu.VMEM_SHARED((NPAD,), jnp.float32),
          pltpu.SemaphoreType.DMA,
          pltpu.SemaphoreType.DMA,
          pltpu.SemaphoreType.DMA,
          pltpu.SemaphoreType.DMA,
          pltpu.SemaphoreType.DMA,
          pltpu.SemaphoreType.DMA,
      ],
  )
  def sc_kernel(x_hbm, src_hbm, dst_hbm, zr_hbm, zc_hbm, on_hbm,
                agg_out, cnt_out,
                srcl, dstl, rows0, rows1, ones, zb0, zb1, czbuf,
                agg_sh, cnt_sh,
                semg0, semg1, sems, semc, semo0, semo1):
    c = lax.axis_index("c")
    s = lax.axis_index("s")
    wid = s * NC + c

    pltpu.async_copy(zr_hbm, zb0, semo0)
    pltpu.async_copy(zc_hbm, czbuf, semo1)
    pltpu.sync_copy(on_hbm, ones)
    pltpu.make_async_copy(zr_hbm, zb0, semo0).wait()
    pltpu.make_async_copy(zc_hbm, czbuf, semo1).wait()

    base = s * RPT
    for k in range(NZB):
      pltpu.sync_copy(zb0, agg_sh.at[pl.ds(base + k * ZROWS, ZROWS)])
    pltpu.sync_copy(czbuf, cnt_sh.at[pl.ds(base, RPT)])

    pltpu.sync_copy(src_hbm.at[wid], srcl)
    pltpu.sync_copy(dst_hbm.at[wid], dstl)
    plsc.subcore_barrier()

    pltpu.async_copy(x_hbm.at[srcl.at[0]], rows0, semg0)
    pltpu.async_copy(x_hbm.at[srcl.at[1]], rows1, semg1)

    def step(i, rows_b, semg_b):
      pltpu.make_async_copy(x_hbm.at[srcl.at[i]], rows_b, semg_b).wait()
      sa = pltpu.async_copy(rows_b, agg_sh.at[dstl.at[i]], sems, add=True)
      sc = pltpu.async_copy(ones, cnt_sh.at[dstl.at[i]], semc, add=True)
      sa.wait()
      sc.wait()
      nxt = jnp.minimum(i + 2, NCHUNK - 1)
      pltpu.async_copy(x_hbm.at[srcl.at[nxt]], rows_b, semg_b)

    def chunk_body(i, _):
      @pl.when(i % 2 == 0)
      def _():
        step(i, rows0, semg0)
      @pl.when(i % 2 == 1)
      def _():
        step(i, rows1, semg1)
      return 0
    lax.fori_loop(0, NCHUNK, chunk_body, 0)
    pltpu.make_async_copy(x_hbm.at[srcl.at[NCHUNK - 1]], rows0, semg0).wait()
    pltpu.make_async_copy(x_hbm.at[srcl.at[NCHUNK - 1]], rows1, semg1).wait()
    plsc.subcore_barrier()

    def oslice(k):
      return pl.ds(base + k * ZROWS, ZROWS)
    zbufs = [zb0, zb1]
    osems = [semo0, semo1]
    for k in range(NZB):
      b = k % 2
      if k >= 2:
        pltpu.make_async_copy(
            zbufs[b], agg_out.at[c, oslice(k - 2)], osems[b]).wait()
      pltpu.sync_copy(agg_sh.at[oslice(k)], zbufs[b])
      pltpu.async_copy(zbufs[b], agg_out.at[c, oslice(k)], osems[b])
    pltpu.sync_copy(cnt_sh.at[pl.ds(base, RPT)], czbuf)
    pltpu.sync_copy(czbuf, cnt_out.at[c, pl.ds(base, RPT)])
    for k in range(max(0, NZB - 2), NZB):
      b = k % 2
      pltpu.make_async_copy(
          zbufs[b], agg_out.at[c, oslice(k)], osems[b]).wait()

  return sc_kernel(xb, src3, dst3, zrows_c, zcnt_c, ones_c)


def _tc_finish_body(agg_ref, cnt_ref, x_ref, wl_ref, bl_ref, wr_ref,
                    g_ref, b_ref, out_ref):
  agg = (agg_ref[0, :N].astype(jnp.float32)
         + agg_ref[1, :N].astype(jnp.float32))
  cnt = cnt_ref[0, :N] + cnt_ref[1, :N]
  mean = agg / jnp.clip(cnt, 1.0, None)[:, None]
  h = (jnp.dot(mean, wl_ref[...].T, preferred_element_type=jnp.float32)
       + bl_ref[...][None, :]
       + jnp.dot(x_ref[...], wr_ref[...].T, preferred_element_type=jnp.float32))
  h = jnp.maximum(h, 0.0)
  mu = jnp.mean(h, axis=0)
  var = jnp.mean((h - mu[None, :]) ** 2, axis=0)
  out_ref[...] = ((h - mu[None, :]) * lax.rsqrt(var + 1e-5)
                  * g_ref[...][None, :] + b_ref[...][None, :])


def kernel(x, edge_index, W_l, b_l, W_r, gamma, beta):
  xb = x.astype(jnp.bfloat16)
  src3 = edge_index[0].reshape(NW, NCHUNK, CHUNK)
  dst3 = edge_index[1].reshape(NW, NCHUNK, CHUNK)
  zrows_c = jnp.zeros((ZROWS, D), jnp.bfloat16)
  zcnt_c = jnp.zeros((RPT,), jnp.float32)
  ones_c = jnp.ones((CHUNK,), jnp.float32)
  agg_p, cnt_p = _sc_aggregate(xb, src3, dst3, zrows_c, zcnt_c, ones_c)
  return pl.pallas_call(
      _tc_finish_body,
      out_shape=jax.ShapeDtypeStruct((N, D), jnp.float32),
  )(agg_p, cnt_p, x, W_l, b_l, W_r, gamma, beta)


# speedup vs baseline: 1.1809x; 1.0686x over previous
import functools

import jax
import jax.numpy as jnp
from jax import lax
from jax.experimental import pallas as pl
from jax.experimental.pallas import tpu as pltpu
from jax.experimental.pallas import tpu_sc as plsc

N = 10000
E = 320000
D = 128

NC = 2
NS = 16
NW = NC * NS

E_PER_W = E // NW
CHUNK = 80
NCHUNK = E_PER_W // CHUNK
NPAD = 10240
RPT = NPAD // NS
ZROWS = 128
NZB = RPT // ZROWS


def _sc_aggregate(xb, src3, dst3, zrows_c, zcnt_c, ones_c):
  mesh = plsc.VectorSubcoreMesh(core_axis_name="c", subcore_axis_name="s")

  @functools.partial(
      pl.kernel,
      out_type=(
          jax.ShapeDtypeStruct((NC, NPAD, D), jnp.bfloat16),
          jax.ShapeDtypeStruct((NC, NPAD), jnp.float32),
      ),
      mesh=mesh,
      compiler_params=pltpu.CompilerParams(use_tc_tiling_on_sc=False),
      scratch_types=[
          pltpu.VMEM((NCHUNK, CHUNK), jnp.int32),
          pltpu.VMEM((NCHUNK, CHUNK), jnp.int32),
          [pltpu.VMEM((CHUNK, D), jnp.bfloat16) for _ in range(4)],
          pltpu.VMEM((CHUNK,), jnp.float32),
          pltpu.VMEM((ZROWS, D), jnp.bfloat16),
          pltpu.VMEM((ZROWS, D), jnp.bfloat16),
          pltpu.VMEM((RPT,), jnp.float32),
          pltpu.VMEM_SHARED((NPAD, D), jnp.bfloat16),
          pltpu.VMEM_SHARED((NPAD,), jnp.float32),
          [pltpu.SemaphoreType.DMA for _ in range(4)],
          [pltpu.SemaphoreType.DMA for _ in range(4)],
          [pltpu.SemaphoreType.DMA for _ in range(4)],
          pltpu.SemaphoreType.DMA,
          pltpu.SemaphoreType.DMA,
      ],
  )
  def sc_kernel(x_hbm, src_hbm, dst_hbm, zr_hbm, zc_hbm, on_hbm,
                agg_out, cnt_out,
                srcl, dstl, rows, ones, zb0, zb1, czbuf,
                agg_sh, cnt_sh,
                semg, semsa, semsc, semo0, semo1):
    c = lax.axis_index("c")
    s = lax.axis_index("s")
    wid = s * NC + c

    pltpu.async_copy(zr_hbm, zb0, semo0)
    pltpu.async_copy(zc_hbm, czbuf, semo1)
    pltpu.sync_copy(on_hbm, ones)
    pltpu.make_async_copy(zr_hbm, zb0, semo0).wait()
    pltpu.make_async_copy(zc_hbm, czbuf, semo1).wait()

    base = s * RPT
    for k in range(NZB):
      pltpu.sync_copy(zb0, agg_sh.at[pl.ds(base + k * ZROWS, ZROWS)])
    pltpu.sync_copy(czbuf, cnt_sh.at[pl.ds(base, RPT)])

    pltpu.sync_copy(src_hbm.at[wid], srcl)
    pltpu.sync_copy(dst_hbm.at[wid], dstl)
    plsc.subcore_barrier()

    pltpu.async_copy(x_hbm.at[srcl.at[0]], rows[0], semg[0])
    pltpu.async_copy(x_hbm.at[srcl.at[1]], rows[1], semg[1])

    def step(i, b):
      bg = (b + 2) % 4
      pltpu.make_async_copy(x_hbm.at[srcl.at[i]], rows[b], semg[b]).wait()
      pltpu.async_copy(rows[b], agg_sh.at[dstl.at[i]], semsa[b], add=True)
      pltpu.async_copy(ones, cnt_sh.at[dstl.at[i]], semsc[b], add=True)
      @pl.when(i >= 2)
      def _():
        prv = jnp.maximum(i - 2, 0)
        pltpu.make_async_copy(
            rows[bg], agg_sh.at[dstl.at[prv]], semsa[bg]).wait()
        pltpu.make_async_copy(
            ones, cnt_sh.at[dstl.at[prv]], semsc[bg]).wait()
      nxt = jnp.minimum(i + 2, NCHUNK - 1)
      pltpu.async_copy(x_hbm.at[srcl.at[nxt]], rows[bg], semg[bg])

    def chunk_body(i, _):
      for b in range(4):
        @pl.when(i % 4 == b)
        def _():
          step(i, b)
      return 0
    lax.fori_loop(0, NCHUNK, chunk_body, 0)
    for i in (NCHUNK, NCHUNK + 1):
      pltpu.make_async_copy(
          x_hbm.at[srcl.at[0]], rows[i % 4], semg[i % 4]).wait()
    for i in (NCHUNK - 2, NCHUNK - 1):
      pltpu.make_async_copy(
          rows[i % 4], agg_sh.at[dstl.at[0]], semsa[i % 4]).wait()
      pltpu.make_async_copy(
          ones, cnt_sh.at[dstl.at[0]], semsc[i % 4]).wait()
    plsc.subcore_barrier()

    def oslice(k):
      return pl.ds(base + k * ZROWS, ZROWS)
    zbufs = [zb0, zb1]
    osems = [semo0, semo1]
    for k in range(NZB):
      b = k % 2
      if k >= 2:
        pltpu.make_async_copy(
            zbufs[b], agg_out.at[c, oslice(k - 2)], osems[b]).wait()
      pltpu.sync_copy(agg_sh.at[oslice(k)], zbufs[b])
      pltpu.async_copy(zbufs[b], agg_out.at[c, oslice(k)], osems[b])
    pltpu.sync_copy(cnt_sh.at[pl.ds(base, RPT)], czbuf)
    pltpu.sync_copy(czbuf, cnt_out.at[c, pl.ds(base, RPT)])
    for k in range(max(0, NZB - 2), NZB):
      b = k % 2
      pltpu.make_async_copy(
          zbufs[b], agg_out.at[c, oslice(k)], osems[b]).wait()

  return sc_kernel(xb, src3, dst3, zrows_c, zcnt_c, ones_c)


def _tc_finish_body(agg_ref, cnt_ref, x_ref, wl_ref, bl_ref, wr_ref,
                    g_ref, b_ref, out_ref):
  agg = (agg_ref[0, :N].astype(jnp.float32)
         + agg_ref[1, :N].astype(jnp.float32))
  cnt = cnt_ref[0, :N] + cnt_ref[1, :N]
  mean = agg / jnp.clip(cnt, 1.0, None)[:, None]
  h = (jnp.dot(mean, wl_ref[...].T, preferred_element_type=jnp.float32)
       + bl_ref[...][None, :]
       + jnp.dot(x_ref[...], wr_ref[...].T, preferred_element_type=jnp.float32))
  h = jnp.maximum(h, 0.0)
  mu = jnp.mean(h, axis=0)
  var = jnp.mean((h - mu[None, :]) ** 2, axis=0)
  out_ref[...] = ((h - mu[None, :]) * lax.rsqrt(var + 1e-5)
                  * g_ref[...][None, :] + b_ref[...][None, :])


def kernel(x, edge_index, W_l, b_l, W_r, gamma, beta):
  xb = x.astype(jnp.bfloat16)
  src3 = edge_index[0].reshape(NW, NCHUNK, CHUNK)
  dst3 = edge_index[1].reshape(NW, NCHUNK, CHUNK)
  zrows_c = jnp.zeros((ZROWS, D), jnp.bfloat16)
  zcnt_c = jnp.zeros((RPT,), jnp.float32)
  ones_c = jnp.ones((CHUNK,), jnp.float32)
  agg_p, cnt_p = _sc_aggregate(xb, src3, dst3, zrows_c, zcnt_c, ones_c)
  return pl.pallas_call(
      _tc_finish_body,
      out_shape=jax.ShapeDtypeStruct((N, D), jnp.float32),
  )(agg_p, cnt_p, x, W_l, b_l, W_r, gamma, beta)
